# baseline (speedup 1.0000x reference)
import functools

import jax
import jax.numpy as jnp
from jax import lax
from jax.experimental import pallas as pl
from jax.experimental.pallas import tpu as pltpu
from jax.experimental.pallas import tpu_sc as plsc

NUM_CORES = 2
NUM_SUBCORES = 16
NUM_WORKERS = NUM_CORES * NUM_SUBCORES
CHUNK = 64
NBUF = 4
LEAD = 3


@functools.partial(jax.jit, static_argnames=("nchunk", "embed_dim"))
def _sc_lookup(table, idx, *, nchunk, embed_dim):
    b_per_w = nchunk * CHUNK
    total = NUM_WORKERS * b_per_w
    mesh = plsc.VectorSubcoreMesh(core_axis_name="c", subcore_axis_name="s")

    @functools.partial(
        pl.kernel,
        out_type=jax.ShapeDtypeStruct((total, embed_dim), jnp.float32),
        mesh=mesh,
        scratch_types=[
            pltpu.VMEM((nchunk, CHUNK), jnp.int32),
            *[pltpu.VMEM((CHUNK, embed_dim), jnp.float32) for _ in range(NBUF)],
            *[pltpu.SemaphoreType.DMA for _ in range(2 * NBUF)],
        ],
    )
    def body(table_hbm, idx_hbm, out_hbm, idx_v, *rest):
        bufs = rest[:NBUF]
        gsems = rest[NBUF : 2 * NBUF]
        ssems = rest[2 * NBUF :]
        wid = lax.axis_index("s") * NUM_CORES + lax.axis_index("c")
        base = wid * b_per_w
        pltpu.sync_copy(idx_hbm.at[wid], idx_v)

        def out_slice(j):
            return out_hbm.at[pl.ds(base + j * CHUNK, CHUNK)]

        def start_gather(j, b):
            pltpu.async_copy(table_hbm.at[idx_v.at[j]], bufs[b], gsems[b])

        def wait_gather(j, b):
            pltpu.make_async_copy(table_hbm.at[idx_v.at[j]], bufs[b], gsems[b]).wait()

        def start_store(j, b):
            pltpu.async_copy(bufs[b], out_hbm.at[idx_v.at[j]], ssems[b])

        def wait_store(j, b):
            pltpu.make_async_copy(bufs[b], out_hbm.at[idx_v.at[j]], ssems[b]).wait()

        for t in range(LEAD):
            start_gather(t, t)

        def step(i, carry):
            for b in range(NBUF):
                j = NBUF * i + b
                ahead = j + LEAD
                nb = (b + LEAD) % NBUF

                @pl.when(ahead < nchunk)
                def _issue():
                    @pl.when(ahead >= NBUF)
                    def _drain():
                        wait_store(ahead - NBUF, nb)

                    start_gather(ahead, nb)

                wait_gather(j, b)
                start_store(j, b)
            return carry

        lax.fori_loop(0, nchunk // NBUF, step, 0)
        for t in range(NBUF):
            j = nchunk - NBUF + t
            wait_store(j, j % NBUF)

    return body(table, idx)


def kernel(x, table):
    embed_dim = table.shape[1]
    xf = x.reshape(-1).astype(jnp.int32)
    b = xf.shape[0]
    grain = NUM_WORKERS * CHUNK * NBUF
    b_pad = ((b + grain - 1) // grain) * grain
    if b_pad != b:
        xf = jnp.pad(xf, (0, b_pad - b))
    nchunk = b_pad // (NUM_WORKERS * CHUNK)
    idx = xf.reshape(NUM_WORKERS, nchunk, CHUNK)
    out = _sc_lookup(table, idx, nchunk=nchunk, embed_dim=embed_dim)
    if b_pad != b:
        out = out[:b]
    return out.reshape(x.shape + (embed_dim,))


# speedup vs baseline: 1.0003x; 1.0003x over previous
import functools

import jax
import jax.numpy as jnp
from jax import lax
from jax.experimental import pallas as pl
from jax.experimental.pallas import tpu as pltpu
from jax.experimental.pallas import tpu_sc as plsc

NUM_CORES = 2
NUM_SUBCORES = 16
NUM_WORKERS = NUM_CORES * NUM_SUBCORES
CHUNK = 64
NBUF = 4
LEAD = 3


@functools.partial(jax.jit, static_argnames=("nchunk", "embed_dim"))
def _sc_lookup(table, idx, *, nchunk, embed_dim):
    b_per_w = nchunk * CHUNK
    total = NUM_WORKERS * b_per_w
    mesh = plsc.VectorSubcoreMesh(core_axis_name="c", subcore_axis_name="s")

    @functools.partial(
        pl.kernel,
        out_type=jax.ShapeDtypeStruct((total, embed_dim), jnp.float32),
        mesh=mesh,
        scratch_types=[
            pltpu.VMEM((nchunk, CHUNK), jnp.int32),
            *[pltpu.VMEM((CHUNK, embed_dim), jnp.float32) for _ in range(NBUF)],
            *[pltpu.SemaphoreType.DMA for _ in range(2 * NBUF)],
        ],
    )
    def body(table_hbm, idx_hbm, out_hbm, idx_v, *rest):
        bufs = rest[:NBUF]
        gsems = rest[NBUF : 2 * NBUF]
        ssems = rest[2 * NBUF :]
        wid = lax.axis_index("s") * NUM_CORES + lax.axis_index("c")
        base = wid * b_per_w
        pltpu.sync_copy(idx_hbm.at[wid], idx_v)

        def out_slice(j):
            return out_hbm.at[pl.ds(base + j * CHUNK, CHUNK)]

        def start_gather(j, b):
            pltpu.async_copy(table_hbm.at[idx_v.at[j]], bufs[b], gsems[b])

        def wait_gather(j, b):
            pltpu.make_async_copy(table_hbm.at[idx_v.at[j]], bufs[b], gsems[b]).wait()

        def start_store(j, b):
            pltpu.async_copy(bufs[b], out_slice(j), ssems[b])

        def wait_store(j, b):
            pltpu.make_async_copy(bufs[b], out_slice(j), ssems[b]).wait()

        for t in range(LEAD):
            start_gather(t, t)

        def step(i, carry):
            for b in range(NBUF):
                j = NBUF * i + b
                ahead = j + LEAD
                nb = (b + LEAD) % NBUF

                @pl.when(ahead < nchunk)
                def _issue():
                    @pl.when(ahead >= NBUF)
                    def _drain():
                        wait_store(ahead - NBUF, nb)

                    start_gather(ahead, nb)

                wait_gather(j, b)
                start_store(j, b)
            return carry

        lax.fori_loop(0, nchunk // NBUF, step, 0)
        for t in range(NBUF):
            j = nchunk - NBUF + t
            wait_store(j, j % NBUF)

    return body(table, idx)


def kernel(x, table):
    embed_dim = table.shape[1]
    xf = x.reshape(-1).astype(jnp.int32)
    b = xf.shape[0]
    grain = NUM_WORKERS * CHUNK * NBUF
    b_pad = ((b + grain - 1) // grain) * grain
    if b_pad != b:
        xf = jnp.pad(xf, (0, b_pad - b))
    nchunk = b_pad // (NUM_WORKERS * CHUNK)
    idx = xf.reshape(NUM_WORKERS, nchunk, CHUNK)
    out = _sc_lookup(table, idx, nchunk=nchunk, embed_dim=embed_dim)
    if b_pad != b:
        out = out[:b]
    return out.reshape(x.shape + (embed_dim,))
